# baseline (speedup 1.0000x reference)
import jax
import jax.numpy as jnp
from jax import lax
from jax.experimental import pallas as pl
from jax.experimental.pallas import tpu as pltpu
from jax.experimental.pallas import tpu_sc as plsc

CAP = 100000
D = 128
B = 16384

NC = 2
NS = 16
NW = NC * NS
L = 16

W_MAIN = 3128
W_LAST = CAP - (NW - 1) * W_MAIN
ROWS_PAD = 3136
MCAP = 2048
CHUNK = 128

CROWS = 136
NCH_COPY = 23
NBUF = 4
RA = 2

FILT_STEPS = B // L
FILT_END = 16
FILT_PER_CH = FILT_STEPS // FILT_END
WSET_END = 20
COMP_STEPS = ROWS_PAD // L
COMP_PER_CH = -(-COMP_STEPS // (NCH_COPY - WSET_END))


def _sc_body(mem_hbm, idx_hbm, val_hbm, out_hbm,
             idx_v, winner_v, mpos_v, mrow_v, fpos_v, frow_v,
             rowbuf_v, cbuf_v, in_sems, out_sems, gat_sems, sct_sems):
    wid = lax.axis_index("s") * NC + lax.axis_index("c")
    lo = wid * W_MAIN
    size = jnp.where(wid == NW - 1, W_LAST, W_MAIN)

    lane = lax.iota(jnp.int32, L)
    neg1 = jnp.full((L,), -1, jnp.int32)
    ones = jnp.full((L,), 1, jnp.int32)
    zeros = jnp.full((L,), 0, jnp.int32)
    lov = jnp.full((L,), lo, jnp.int32)
    szv = jnp.full((L,), size, jnp.int32)

    def off(k):
        return jnp.minimum(k * CROWS, size - CROWS)

    def copy_in(k):
        return pltpu.make_async_copy(
            mem_hbm.at[pl.ds(lo + off(k), CROWS)],
            cbuf_v.at[k % NBUF],
            in_sems.at[k % NBUF],
        )

    def copy_out(k):
        return pltpu.make_async_copy(
            cbuf_v.at[k % NBUF],
            out_hbm.at[pl.ds(lo + off(k), CROWS)],
            out_sems.at[k % NBUF],
        )

    pltpu.sync_copy(idx_hbm, idx_v)

    def prefix_sum_excl(mi):
        return plsc.cumsum(mi) - mi

    def filt(i, cnt):
        v = idx_v[pl.ds(i * L, L)]
        local = v - lov
        m = (local >= 0) & (local < szv)
        mi = jnp.where(m, ones, zeros)
        dest = cnt + prefix_sum_excl(mi)
        plsc.store_scatter(mpos_v, [dest], i * L + lane, mask=m)
        plsc.store_scatter(mrow_v, [dest], local, mask=m)
        pc = plsc.all_reduce_population_count(m)
        return cnt + pc[0]

    def winit(i, _):
        winner_v[pl.ds(i * L, L)] = neg1
        return 0

    lax.fori_loop(0, ROWS_PAD // L, winit, 0)

    def wset(g, _):
        rows = mrow_v[pl.ds(g * L, L)]
        poss = mpos_v[pl.ds(g * L, L)]
        valid = rows >= 0
        for k in range(L):
            plsc.store_scatter(winner_v, [rows], poss,
                               mask=valid & (lane == k))
        return 0

    def compact(i, cnt2):
        w = winner_v[pl.ds(i * L, L)]
        m = w >= 0
        mi = jnp.where(m, ones, zeros)
        dest = cnt2 + prefix_sum_excl(mi)
        plsc.store_scatter(fpos_v, [dest], w, mask=m)
        plsc.store_scatter(frow_v, [dest], lo + i * L + lane, mask=m)
        pc = plsc.all_reduce_population_count(m)
        return cnt2 + pc[0]

    for k in range(RA):
        copy_in(k).start()

    def do_filt(k, cnt, cnt2):
        cnt = lax.fori_loop(k * FILT_PER_CH, (k + 1) * FILT_PER_CH,
                            filt, cnt)

        @pl.when(k == FILT_END - 1)
        def _():
            mpos_v[pl.ds(cnt, L)] = neg1
            mrow_v[pl.ds(cnt, L)] = neg1

        return (cnt, cnt2)

    def do_wset(k, cnt, cnt2):
        gmax = (cnt + L - 1) // L
        j = k - FILT_END
        gper = (gmax + (WSET_END - FILT_END) - 1) // (WSET_END - FILT_END)
        lax.fori_loop(jnp.minimum(j * gper, gmax),
                      jnp.minimum((j + 1) * gper, gmax), wset, 0)
        return (cnt, cnt2)

    def do_compact(k, cnt, cnt2):
        j = k - WSET_END
        c0 = jnp.minimum(j * COMP_PER_CH, COMP_STEPS)
        c1 = jnp.minimum((j + 1) * COMP_PER_CH, COMP_STEPS)
        cnt2 = lax.fori_loop(c0, c1, compact, cnt2)
        return (cnt, cnt2)

    def ring_step(k, carry):
        cnt, cnt2 = carry
        copy_in(k).wait()
        copy_out(k).start()

        cnt, cnt2 = lax.cond(
            k < FILT_END, do_filt,
            lambda k, cnt, cnt2: lax.cond(
                k < WSET_END, do_wset, do_compact, k, cnt, cnt2),
            k, cnt, cnt2)

        @pl.when((k >= RA) & (k + RA < NCH_COPY))
        def _():
            copy_out(k - RA).wait()

        @pl.when(k + RA < NCH_COPY)
        def _():
            copy_in(k + RA).start()

        return (cnt, cnt2)

    _, nfinal = lax.fori_loop(0, NCH_COPY, ring_step,
                              (jnp.int32(0), jnp.int32(0)))

    def tailfill(t, _):
        fpos_v[pl.ds(nfinal + t * L, L)] = neg1
        frow_v[pl.ds(nfinal + t * L, L)] = neg1
        return 0

    lax.fori_loop(0, CHUNK // L, tailfill, 0)

    nch = (nfinal + CHUNK - 1) // CHUNK

    def gather(c):
        b = c % 2
        gpos = plsc.Indices(fpos_v.at[pl.ds(c * CHUNK, CHUNK)],
                            ignored_value=-1)
        return pltpu.make_async_copy(val_hbm.at[gpos], rowbuf_v.at[b],
                                     gat_sems.at[b])

    def scatter(c):
        b = c % 2
        grow = plsc.Indices(frow_v.at[pl.ds(c * CHUNK, CHUNK)],
                            ignored_value=-1)
        return pltpu.make_async_copy(rowbuf_v.at[b], out_hbm.at[grow],
                                     sct_sems.at[b])

    @pl.when(nch > 0)
    def _():
        gather(0).start()

    def drain(k, _):
        copy_out(k).wait()
        return 0

    lax.fori_loop(NCH_COPY - 2 * RA, NCH_COPY, drain, 0)

    def scat(c, _):
        gather(c).wait()
        scatter(c).start()

        @pl.when(c > 0)
        def _():
            scatter(c - 1).wait()

        @pl.when(c + 1 < nch)
        def _():
            gather(c + 1).start()

        return 0

    lax.fori_loop(0, nch, scat, 0)

    @pl.when(nch > 0)
    def _():
        scatter(nch - 1).wait()


@jax.jit
def _scatter_sc(mem, idx, val):
    mesh = plsc.VectorSubcoreMesh(
        core_axis_name="c", subcore_axis_name="s",
        num_cores=NC, num_subcores=NS,
    )
    return pl.kernel(
        _sc_body,
        out_type=jax.ShapeDtypeStruct((CAP, D), jnp.float32),
        mesh=mesh,
        compiler_params=pltpu.CompilerParams(needs_layout_passes=False),
        scratch_types=[
            pltpu.VMEM((B,), jnp.int32),
            pltpu.VMEM((ROWS_PAD,), jnp.int32),
            pltpu.VMEM((MCAP,), jnp.int32),
            pltpu.VMEM((MCAP,), jnp.int32),
            pltpu.VMEM((MCAP + CHUNK,), jnp.int32),
            pltpu.VMEM((MCAP + CHUNK,), jnp.int32),
            pltpu.VMEM((2, CHUNK, D), jnp.float32),
            pltpu.VMEM((NBUF, CROWS, D), jnp.float32),
            pltpu.SemaphoreType.DMA((NBUF,)),
            pltpu.SemaphoreType.DMA((NBUF,)),
            pltpu.SemaphoreType.DMA((2,)),
            pltpu.SemaphoreType.DMA((2,)),
        ],
    )(mem, idx, val)


def kernel(mem, idx, val):
    return _scatter_sc(mem, idx, val)


# speedup vs baseline: 1.0332x; 1.0332x over previous
import jax
import jax.numpy as jnp
from jax import lax
from jax.experimental import pallas as pl
from jax.experimental.pallas import tpu as pltpu
from jax.experimental.pallas import tpu_sc as plsc

CAP = 100000
D = 128
B = 16384

NC = 2
NS = 16
NW = NC * NS
L = 16

W_MAIN = 3128
W_LAST = CAP - (NW - 1) * W_MAIN
ROWS_PAD = 3136
MCAP = 2048
CHUNK = 128

CROWS = 136
NCH_COPY = 23
NBUF = 4
RA = 3

FILT_STEPS = B // L
FILT_END = 16
FILT_PER_CH = FILT_STEPS // FILT_END
WSET_END = 20
COMP_STEPS = ROWS_PAD // L
COMP_PER_CH = -(-COMP_STEPS // (NCH_COPY - WSET_END))


def _sc_body(mem_hbm, idx_hbm, val_hbm, out_hbm,
             idx_v, winner_v, mpos_v, mrow_v, fpos_v, frow_v,
             rowbuf_v, cbuf_v, in_sems, out_sems, gat_sems, sct_sems):
    wid = lax.axis_index("s") * NC + lax.axis_index("c")
    lo = wid * W_MAIN
    size = jnp.where(wid == NW - 1, W_LAST, W_MAIN)

    lane = lax.iota(jnp.int32, L)
    neg1 = jnp.full((L,), -1, jnp.int32)
    ones = jnp.full((L,), 1, jnp.int32)
    zeros = jnp.full((L,), 0, jnp.int32)
    lov = jnp.full((L,), lo, jnp.int32)
    szv = jnp.full((L,), size, jnp.int32)

    def off(k):
        return jnp.minimum(k * CROWS, size - CROWS)

    def copy_in(k):
        return pltpu.make_async_copy(
            mem_hbm.at[pl.ds(lo + off(k), CROWS)],
            cbuf_v.at[k % NBUF],
            in_sems.at[k % NBUF],
        )

    def copy_out(k):
        return pltpu.make_async_copy(
            cbuf_v.at[k % NBUF],
            out_hbm.at[pl.ds(lo + off(k), CROWS)],
            out_sems.at[k % NBUF],
        )

    pltpu.sync_copy(idx_hbm, idx_v)

    def prefix_sum_excl(mi):
        return plsc.cumsum(mi) - mi

    def filt(i, cnt):
        v = idx_v[pl.ds(i * L, L)]
        local = v - lov
        m = (local >= 0) & (local < szv)
        mi = jnp.where(m, ones, zeros)
        dest = cnt + prefix_sum_excl(mi)
        plsc.store_scatter(mpos_v, [dest], i * L + lane, mask=m)
        plsc.store_scatter(mrow_v, [dest], local, mask=m)
        pc = plsc.all_reduce_population_count(m)
        return cnt + pc[0]

    def winit(i, _):
        winner_v[pl.ds(i * L, L)] = neg1
        return 0

    lax.fori_loop(0, ROWS_PAD // L, winit, 0)

    def wset(g, _):
        rows = mrow_v[pl.ds(g * L, L)]
        poss = mpos_v[pl.ds(g * L, L)]
        valid = rows >= 0
        for k in range(L):
            plsc.store_scatter(winner_v, [rows], poss,
                               mask=valid & (lane == k))
        return 0

    def compact(i, cnt2):
        w = winner_v[pl.ds(i * L, L)]
        m = w >= 0
        mi = jnp.where(m, ones, zeros)
        dest = cnt2 + prefix_sum_excl(mi)
        plsc.store_scatter(fpos_v, [dest], w, mask=m)
        plsc.store_scatter(frow_v, [dest], lo + i * L + lane, mask=m)
        pc = plsc.all_reduce_population_count(m)
        return cnt2 + pc[0]

    for k in range(RA):
        copy_in(k).start()

    def do_filt(k, cnt, cnt2):
        cnt = lax.fori_loop(k * FILT_PER_CH, (k + 1) * FILT_PER_CH,
                            filt, cnt)

        @pl.when(k == FILT_END - 1)
        def _():
            mpos_v[pl.ds(cnt, L)] = neg1
            mrow_v[pl.ds(cnt, L)] = neg1

        return (cnt, cnt2)

    def do_wset(k, cnt, cnt2):
        gmax = (cnt + L - 1) // L
        j = k - FILT_END
        gper = (gmax + (WSET_END - FILT_END) - 1) // (WSET_END - FILT_END)
        lax.fori_loop(jnp.minimum(j * gper, gmax),
                      jnp.minimum((j + 1) * gper, gmax), wset, 0)
        return (cnt, cnt2)

    def do_compact(k, cnt, cnt2):
        j = k - WSET_END
        c0 = jnp.minimum(j * COMP_PER_CH, COMP_STEPS)
        c1 = jnp.minimum((j + 1) * COMP_PER_CH, COMP_STEPS)
        cnt2 = lax.fori_loop(c0, c1, compact, cnt2)
        return (cnt, cnt2)

    def ring_step(k, carry):
        cnt, cnt2 = carry
        copy_in(k).wait()
        copy_out(k).start()

        cnt, cnt2 = lax.cond(
            k < FILT_END, do_filt,
            lambda k, cnt, cnt2: lax.cond(
                k < WSET_END, do_wset, do_compact, k, cnt, cnt2),
            k, cnt, cnt2)

        @pl.when((k + RA >= NBUF) & (k + RA < NCH_COPY))
        def _():
            copy_out(k + RA - NBUF).wait()

        @pl.when(k + RA < NCH_COPY)
        def _():
            copy_in(k + RA).start()

        return (cnt, cnt2)

    _, nfinal = lax.fori_loop(0, NCH_COPY, ring_step,
                              (jnp.int32(0), jnp.int32(0)))

    def tailfill(t, _):
        fpos_v[pl.ds(nfinal + t * L, L)] = neg1
        frow_v[pl.ds(nfinal + t * L, L)] = neg1
        return 0

    lax.fori_loop(0, CHUNK // L, tailfill, 0)

    nch = (nfinal + CHUNK - 1) // CHUNK

    def gather(c):
        b = c % 2
        gpos = plsc.Indices(fpos_v.at[pl.ds(c * CHUNK, CHUNK)],
                            ignored_value=-1)
        return pltpu.make_async_copy(val_hbm.at[gpos], rowbuf_v.at[b],
                                     gat_sems.at[b])

    def scatter(c):
        b = c % 2
        grow = plsc.Indices(frow_v.at[pl.ds(c * CHUNK, CHUNK)],
                            ignored_value=-1)
        return pltpu.make_async_copy(rowbuf_v.at[b], out_hbm.at[grow],
                                     sct_sems.at[b])

    @pl.when(nch > 0)
    def _():
        gather(0).start()

    def drain(k, _):
        copy_out(k).wait()
        return 0

    lax.fori_loop(NCH_COPY - NBUF, NCH_COPY, drain, 0)

    def scat(c, _):
        gather(c).wait()
        scatter(c).start()

        @pl.when(c > 0)
        def _():
            scatter(c - 1).wait()

        @pl.when(c + 1 < nch)
        def _():
            gather(c + 1).start()

        return 0

    lax.fori_loop(0, nch, scat, 0)

    @pl.when(nch > 0)
    def _():
        scatter(nch - 1).wait()


@jax.jit
def _scatter_sc(mem, idx, val):
    mesh = plsc.VectorSubcoreMesh(
        core_axis_name="c", subcore_axis_name="s",
        num_cores=NC, num_subcores=NS,
    )
    return pl.kernel(
        _sc_body,
        out_type=jax.ShapeDtypeStruct((CAP, D), jnp.float32),
        mesh=mesh,
        compiler_params=pltpu.CompilerParams(needs_layout_passes=False),
        scratch_types=[
            pltpu.VMEM((B,), jnp.int32),
            pltpu.VMEM((ROWS_PAD,), jnp.int32),
            pltpu.VMEM((MCAP,), jnp.int32),
            pltpu.VMEM((MCAP,), jnp.int32),
            pltpu.VMEM((MCAP + CHUNK,), jnp.int32),
            pltpu.VMEM((MCAP + CHUNK,), jnp.int32),
            pltpu.VMEM((2, CHUNK, D), jnp.float32),
            pltpu.VMEM((NBUF, CROWS, D), jnp.float32),
            pltpu.SemaphoreType.DMA((NBUF,)),
            pltpu.SemaphoreType.DMA((NBUF,)),
            pltpu.SemaphoreType.DMA((2,)),
            pltpu.SemaphoreType.DMA((2,)),
        ],
    )(mem, idx, val)


def kernel(mem, idx, val):
    return _scatter_sc(mem, idx, val)
